# baseline (speedup 1.0000x reference)
import functools

import jax
import jax.numpy as jnp
from jax import lax
from jax.experimental import pallas as pl
from jax.experimental.pallas import tpu as pltpu
from jax.experimental.pallas import tpu_sc as plsc

E = 8
K = 2
D = 1024
F = 4096
T = 8192
TM = 256
NP = T * K + E * TM
NTILES = NP // TM
TILE_T = 512
NT = T // TILE_T
LANES = 128
NW = 32
CB = 64
CD = 16
TPW = T // NW


def _router_body(x_ref, wr_ref, pos_ref, g0_ref, g1_ref, te_ref, cnt_ref,
                 off_ref):
    p = pl.program_id(0)
    i = pl.program_id(1)

    @pl.when(jnp.logical_and(p == 0, i == 0))
    def _init():
        cnt_ref[...] = jnp.zeros_like(cnt_ref)

    @pl.when(jnp.logical_and(p == 1, i == 0))
    def _offsets():
        counts = cnt_ref[0:1, :]
        padded = jnp.floor((counts + (TM - 1)) * (1.0 / TM)) * TM
        ci = lax.broadcasted_iota(jnp.int32, (LANES, LANES), 0)
        cj = lax.broadcasted_iota(jnp.int32, (LANES, LANES), 1)
        upper = (ci < cj).astype(jnp.float32)
        off = jax.lax.dot_general(
            padded, upper, (((1,), (0,)), ((), ())),
            precision=jax.lax.Precision.HIGHEST,
            preferred_element_type=jnp.float32)
        ends = off + padded
        ones_col = jnp.ones((LANES, 1), jnp.float32)
        ends_bc = jax.lax.dot_general(
            ones_col, ends, (((1,), (0,)), ((), ())),
            precision=jax.lax.Precision.HIGHEST,
            preferred_element_type=jnp.float32)
        jrow = (lax.broadcasted_iota(jnp.int32, (LANES, LANES), 0) * TM
                ).astype(jnp.float32)
        ecol = lax.broadcasted_iota(jnp.int32, (LANES, LANES), 1)
        ind = jnp.where((ends_bc <= jrow) & (ecol < E), 1.0, 0.0)
        texp = jnp.minimum(jnp.sum(ind, axis=1, keepdims=True), E - 1.0)
        te_ref[...] = jnp.broadcast_to(texp + 0.5, (LANES, LANES)).astype(jnp.int32)
        off_ref[...] = jnp.broadcast_to(off, (8, LANES))
        cnt_ref[...] = jnp.zeros_like(cnt_ref)

    x = x_ref[...]
    wr = wr_ref[...]
    logits = jnp.dot(x, wr, preferred_element_type=jnp.float32)
    lane = lax.broadcasted_iota(jnp.int32, (TILE_T, LANES), 1)
    valid = lane < E
    lm = jnp.where(valid, logits, -1e30)
    m = jnp.max(lm, axis=1, keepdims=True)
    ez = jnp.where(valid, jnp.exp(lm - m), 0.0)
    probs = ez / jnp.sum(ez, axis=1, keepdims=True)
    p1 = jnp.max(probs, axis=1, keepdims=True)
    i1 = jnp.min(jnp.where((probs == p1) & valid, lane, LANES), axis=1, keepdims=True)
    oh0 = (lane == i1)
    pr2 = jnp.where(oh0, -1.0, probs)
    p2 = jnp.max(pr2, axis=1, keepdims=True)
    i2 = jnp.min(jnp.where((pr2 == p2) & valid, lane, LANES), axis=1, keepdims=True)
    oh1 = (lane == i2)
    s = p1 + p2 + 1e-9
    g0 = p1 / s
    g1 = p2 / s

    oh0f = oh0.astype(jnp.float32)
    oh1f = oh1.astype(jnp.float32)
    ri = lax.broadcasted_iota(jnp.int32, (TILE_T, TILE_T), 0)
    rj = lax.broadcasted_iota(jnp.int32, (TILE_T, TILE_T), 1)
    ltri = (ri >= rj).astype(jnp.float32)
    c0 = jnp.dot(ltri, oh0f, preferred_element_type=jnp.float32)
    c1 = jnp.dot(ltri, oh1f, preferred_element_type=jnp.float32)
    t0 = c0[TILE_T - 1:TILE_T, :]
    t1 = c1[TILE_T - 1:TILE_T, :]
    cnt_row = cnt_ref[0:1, :]

    @pl.when(p == 1)
    def _emit():
        off_row = off_ref[0:1, :]
        c0_at = jnp.sum(c0 * oh0f, axis=1, keepdims=True)
        c1_at = jnp.sum(c1 * oh1f, axis=1, keepdims=True)
        cnt0 = jnp.sum(cnt_row * oh0f, axis=1, keepdims=True)
        cnt1 = jnp.sum(cnt_row * oh1f, axis=1, keepdims=True)
        t0_at1 = jnp.sum(t0 * oh1f, axis=1, keepdims=True)
        off0 = jnp.sum(off_row * oh0f, axis=1, keepdims=True)
        off1 = jnp.sum(off_row * oh1f, axis=1, keepdims=True)
        pos0 = off0 + cnt0 + c0_at - 1.0
        pos1 = off1 + cnt1 + t0_at1 + c1_at - 1.0
        posv = jnp.where(lane < 64, pos0, pos1)
        pos_ref[...] = (posv + 0.5).astype(jnp.int32)
        g0_ref[...] = jnp.broadcast_to(g0, (TILE_T, LANES))
        g1_ref[...] = jnp.broadcast_to(g1, (TILE_T, LANES))

    cnt_ref[...] = jnp.broadcast_to(cnt_row + t0 + t1, (8, LANES))


def _router_call(x, wrp):
    return pl.pallas_call(
        _router_body,
        grid=(2, NT),
        in_specs=[
            pl.BlockSpec((TILE_T, D), lambda p, i: (i, 0)),
            pl.BlockSpec((D, LANES), lambda p, i: (0, 0)),
        ],
        out_specs=[
            pl.BlockSpec((TILE_T, LANES), lambda p, i: (i, 0)),
            pl.BlockSpec((TILE_T, LANES), lambda p, i: (i, 0)),
            pl.BlockSpec((TILE_T, LANES), lambda p, i: (i, 0)),
            pl.BlockSpec((LANES, LANES), lambda p, i: (0, 0)),
        ],
        out_shape=[
            jax.ShapeDtypeStruct((T, LANES), jnp.int32),
            jax.ShapeDtypeStruct((T, LANES), jnp.float32),
            jax.ShapeDtypeStruct((T, LANES), jnp.float32),
            jax.ShapeDtypeStruct((LANES, LANES), jnp.int32),
        ],
        scratch_shapes=[
            pltpu.VMEM((8, LANES), jnp.float32),
            pltpu.VMEM((8, LANES), jnp.float32),
        ],
        compiler_params=pltpu.CompilerParams(
            dimension_semantics=("arbitrary", "arbitrary")),
    )(x, wrp)


def _ffn_body(te_ref, xs_ref, w1_ref, b1_ref, w2_ref, b2_ref, y_ref):
    h = jnp.dot(xs_ref[...].astype(jnp.bfloat16), w1_ref[0],
                preferred_element_type=jnp.float32) + b1_ref[0]
    h = jax.nn.gelu(h)
    y = jnp.dot(h.astype(jnp.bfloat16), w2_ref[0],
                preferred_element_type=jnp.float32) + b2_ref[0]
    y_ref[...] = y


def _ffn_call(tile_expert, xs, w1b, b1r, w2b, b2r):
    grid_spec = pltpu.PrefetchScalarGridSpec(
        num_scalar_prefetch=1,
        grid=(NTILES,),
        in_specs=[
            pl.BlockSpec((TM, D), lambda i, te: (i, 0)),
            pl.BlockSpec((1, D, F), lambda i, te: (te[i], 0, 0)),
            pl.BlockSpec((1, 1, F), lambda i, te: (te[i], 0, 0)),
            pl.BlockSpec((1, F, D), lambda i, te: (te[i], 0, 0)),
            pl.BlockSpec((1, 1, D), lambda i, te: (te[i], 0, 0)),
        ],
        out_specs=pl.BlockSpec((TM, D), lambda i, te: (i, 0)),
    )
    return pl.pallas_call(
        _ffn_body,
        grid_spec=grid_spec,
        out_shape=jax.ShapeDtypeStruct((NP, D), jnp.float32),
    )(tile_expert, xs, w1b, b1r, w2b, b2r)


def _sc_mesh():
    return plsc.VectorSubcoreMesh(core_axis_name="c", subcore_axis_name="s")


def _scatter_call(x, pos0r, pos1r):
    @functools.partial(
        pl.kernel,
        mesh=_sc_mesh(),
        out_type=jax.ShapeDtypeStruct((NP, D), jnp.float32),
        scratch_types=[
            pltpu.VMEM((CB, D), jnp.float32),
            pltpu.VMEM((CB,), jnp.int32),
            pltpu.VMEM((CB,), jnp.int32),
            pltpu.SemaphoreType.DMA,
            pltpu.SemaphoreType.DMA,
        ],
    )
    def scatter_kernel(x_hbm, pos0_hbm, pos1_hbm, xs_hbm,
                       rows_v, idx0_v, idx1_v, sem0, sem1):
        wid = lax.axis_index("s") * 2 + lax.axis_index("c")
        for c in range(TPW // CB):
            base = wid * TPW + c * CB
            pltpu.sync_copy(x_hbm.at[pl.ds(base, CB)], rows_v)
            pltpu.sync_copy(pos0_hbm.at[wid, c], idx0_v)
            pltpu.sync_copy(pos1_hbm.at[wid, c], idx1_v)
            cp0 = pltpu.async_copy(rows_v, xs_hbm.at[idx0_v], sem0)
            cp1 = pltpu.async_copy(rows_v, xs_hbm.at[idx1_v], sem1)
            cp0.wait()
            cp1.wait()

    return scatter_kernel(x, pos0r, pos1r)


def _combine_call(ys, pos0r, pos1r, g0r, g1r):
    ncd = TPW // CD

    @functools.partial(
        pl.kernel,
        mesh=_sc_mesh(),
        out_type=jax.ShapeDtypeStruct((T, D), jnp.float32),
        scratch_types=[
            pltpu.VMEM((2, CD, D), jnp.float32),
            pltpu.VMEM((2, CD, D), jnp.float32),
            pltpu.VMEM((2, CD, D), jnp.float32),
            pltpu.VMEM((2, CD), jnp.int32),
            pltpu.VMEM((2, CD), jnp.int32),
            pltpu.VMEM((CD, LANES), jnp.float32),
            pltpu.VMEM((CD, LANES), jnp.float32),
            pltpu.SemaphoreType.DMA,
            pltpu.SemaphoreType.DMA,
            pltpu.SemaphoreType.DMA,
            pltpu.SemaphoreType.DMA,
        ],
    )
    def combine_kernel(y_hbm, pos0_hbm, pos1_hbm, g0_hbm, g1_hbm, out_hbm,
                       y0_v, y1_v, o_v, idx0_v, idx1_v, g0_v, g1_v,
                       sg0, sg1, so0, so1):
        wid = lax.axis_index("s") * 2 + lax.axis_index("c")
        sg = [sg0, sg1]
        so = [so0, so1]
        handles = {}

        def issue(c):
            b = c % 2
            pltpu.sync_copy(pos0_hbm.at[wid, c], idx0_v.at[b])
            pltpu.sync_copy(pos1_hbm.at[wid, c], idx1_v.at[b])
            handles[(0, c)] = pltpu.async_copy(
                y_hbm.at[idx0_v.at[b]], y0_v.at[b], sg[b])
            handles[(1, c)] = pltpu.async_copy(
                y_hbm.at[idx1_v.at[b]], y1_v.at[b], sg[b])

        issue(0)
        for c in range(ncd):
            b = c % 2
            if c + 1 < ncd:
                issue(c + 1)
            handles[(0, c)].wait()
            handles[(1, c)].wait()
            pltpu.sync_copy(g0_hbm.at[wid, c], g0_v)
            pltpu.sync_copy(g1_hbm.at[wid, c], g1_v)
            if c >= 2:
                handles[("o", c - 2)].wait()

            def row_body(r, carry):
                g0s = g0_v[r, pl.ds(0, 16)]
                g1s = g1_v[r, pl.ds(0, 16)]

                def sl_body(sl, carry2):
                    off = sl * 16
                    o_v[b, r, pl.ds(off, 16)] = (
                        y0_v[b, r, pl.ds(off, 16)] * g0s
                        + y1_v[b, r, pl.ds(off, 16)] * g1s)
                    return carry2

                return lax.fori_loop(0, D // 16, sl_body, carry, unroll=16)

            lax.fori_loop(0, CD, row_body, 0)
            handles[("o", c)] = pltpu.async_copy(
                o_v.at[b], out_hbm.at[pl.ds(wid * TPW + c * CD, CD)], so[b])
        handles[("o", ncd - 2)].wait()
        handles[("o", ncd - 1)].wait()

    return combine_kernel(ys, pos0r, pos1r, g0r, g1r)


def kernel(x, w_router, w1, b1, w2, b2):
    wrp = jnp.pad(w_router, ((0, 0), (0, LANES - E)))
    pos, g0b, g1b, te = _router_call(x, wrp)
    pos0 = pos[:, 0]
    pos1 = pos[:, 64]
    tile_expert = te[:NTILES, 0]
    xs = _scatter_call(
        x,
        pos0.reshape(NW, TPW // CB, CB),
        pos1.reshape(NW, TPW // CB, CB),
    )
    ys = _ffn_call(tile_expert, xs,
                   w1.astype(jnp.bfloat16), b1.reshape(E, 1, F),
                   w2.astype(jnp.bfloat16), b2.reshape(E, 1, D))
    out = _combine_call(
        ys,
        pos0.reshape(NW, TPW // CD, CD),
        pos1.reshape(NW, TPW // CD, CD),
        g0b.reshape(NW, TPW // CD, CD, LANES),
        g1b.reshape(NW, TPW // CD, CD, LANES),
    )
    return out


# speedup vs baseline: 1.0332x; 1.0332x over previous
import functools

import jax
import jax.numpy as jnp
from jax import lax
from jax.experimental import pallas as pl
from jax.experimental.pallas import tpu as pltpu
from jax.experimental.pallas import tpu_sc as plsc

E = 8
K = 2
D = 1024
F = 4096
T = 8192
TM = 256
NP = T * K + E * TM
NTILES = NP // TM
TILE_T = 512
NT = T // TILE_T
LANES = 128
NW = 32
CB = 64
CD = 16
TPW = T // NW


def _router_body(x_ref, wr_ref, pos_ref, gb_ref, te_ref, cnt_ref, off_ref):
    p = pl.program_id(0)
    i = pl.program_id(1)

    @pl.when(jnp.logical_and(p == 0, i == 0))
    def _init():
        cnt_ref[...] = jnp.zeros_like(cnt_ref)

    @pl.when(jnp.logical_and(p == 1, i == 0))
    def _offsets():
        counts = cnt_ref[0:1, :]
        padded = jnp.floor((counts + (TM - 1)) * (1.0 / TM)) * TM
        ci = lax.broadcasted_iota(jnp.int32, (LANES, LANES), 0)
        cj = lax.broadcasted_iota(jnp.int32, (LANES, LANES), 1)
        upper = (ci < cj).astype(jnp.float32)
        off = jax.lax.dot_general(
            padded, upper, (((1,), (0,)), ((), ())),
            precision=jax.lax.Precision.HIGHEST,
            preferred_element_type=jnp.float32)
        ends = off + padded
        ones_col = jnp.ones((LANES, 1), jnp.float32)
        ends_bc = jax.lax.dot_general(
            ones_col, ends, (((1,), (0,)), ((), ())),
            precision=jax.lax.Precision.HIGHEST,
            preferred_element_type=jnp.float32)
        jrow = (lax.broadcasted_iota(jnp.int32, (LANES, LANES), 0) * TM
                ).astype(jnp.float32)
        ecol = lax.broadcasted_iota(jnp.int32, (LANES, LANES), 1)
        ind = jnp.where((ends_bc <= jrow) & (ecol < E), 1.0, 0.0)
        texp = jnp.minimum(jnp.sum(ind, axis=1, keepdims=True), E - 1.0)
        te_ref[...] = jnp.broadcast_to(texp + 0.5, (LANES, LANES)).astype(jnp.int32)
        off_ref[...] = jnp.broadcast_to(off, (8, LANES))
        cnt_ref[...] = jnp.zeros_like(cnt_ref)

    x = x_ref[...]
    wr = wr_ref[...]
    logits = jnp.dot(x, wr, preferred_element_type=jnp.float32)
    lane = lax.broadcasted_iota(jnp.int32, (TILE_T, LANES), 1)
    valid = lane < E
    lm = jnp.where(valid, logits, -1e30)
    m = jnp.max(lm, axis=1, keepdims=True)
    ez = jnp.where(valid, jnp.exp(lm - m), 0.0)
    probs = ez / jnp.sum(ez, axis=1, keepdims=True)
    p1 = jnp.max(probs, axis=1, keepdims=True)
    i1 = jnp.min(jnp.where((probs == p1) & valid, lane, LANES), axis=1, keepdims=True)
    oh0 = (lane == i1)
    pr2 = jnp.where(oh0, -1.0, probs)
    p2 = jnp.max(pr2, axis=1, keepdims=True)
    i2 = jnp.min(jnp.where((pr2 == p2) & valid, lane, LANES), axis=1, keepdims=True)
    oh1 = (lane == i2)
    s = p1 + p2 + 1e-9
    g0 = p1 / s
    g1 = p2 / s

    oh0f = oh0.astype(jnp.float32)
    oh1f = oh1.astype(jnp.float32)
    ri = lax.broadcasted_iota(jnp.int32, (TILE_T, TILE_T), 0)
    rj = lax.broadcasted_iota(jnp.int32, (TILE_T, TILE_T), 1)
    ltri = (ri >= rj).astype(jnp.float32)
    c0 = jnp.dot(ltri, oh0f, preferred_element_type=jnp.float32)
    c1 = jnp.dot(ltri, oh1f, preferred_element_type=jnp.float32)
    t0 = c0[TILE_T - 1:TILE_T, :]
    t1 = c1[TILE_T - 1:TILE_T, :]
    cnt_row = cnt_ref[0:1, :]

    @pl.when(p == 1)
    def _emit():
        off_row = off_ref[0:1, :]
        c0_at = jnp.sum(c0 * oh0f, axis=1, keepdims=True)
        c1_at = jnp.sum(c1 * oh1f, axis=1, keepdims=True)
        cnt0 = jnp.sum(cnt_row * oh0f, axis=1, keepdims=True)
        cnt1 = jnp.sum(cnt_row * oh1f, axis=1, keepdims=True)
        t0_at1 = jnp.sum(t0 * oh1f, axis=1, keepdims=True)
        off0 = jnp.sum(off_row * oh0f, axis=1, keepdims=True)
        off1 = jnp.sum(off_row * oh1f, axis=1, keepdims=True)
        pos0 = off0 + cnt0 + c0_at - 1.0
        pos1 = off1 + cnt1 + t0_at1 + c1_at - 1.0
        posv = jnp.where(lane < 64, pos0, pos1)
        pos_ref[...] = (posv + 0.5).astype(jnp.int32)
        gb_ref[...] = jnp.where(lane < 64, g0, g1)

    cnt_ref[...] = jnp.broadcast_to(cnt_row + t0 + t1, (8, LANES))


def _router_call(x, wrp):
    return pl.pallas_call(
        _router_body,
        grid=(2, NT),
        in_specs=[
            pl.BlockSpec((TILE_T, D), lambda p, i: (i, 0)),
            pl.BlockSpec((D, LANES), lambda p, i: (0, 0)),
        ],
        out_specs=[
            pl.BlockSpec((TILE_T, LANES), lambda p, i: (i, 0)),
            pl.BlockSpec((TILE_T, LANES), lambda p, i: (i, 0)),
            pl.BlockSpec((LANES, LANES), lambda p, i: (0, 0)),
        ],
        out_shape=[
            jax.ShapeDtypeStruct((T, LANES), jnp.int32),
            jax.ShapeDtypeStruct((T, LANES), jnp.float32),
            jax.ShapeDtypeStruct((LANES, LANES), jnp.int32),
        ],
        scratch_shapes=[
            pltpu.VMEM((8, LANES), jnp.float32),
            pltpu.VMEM((8, LANES), jnp.float32),
        ],
        compiler_params=pltpu.CompilerParams(
            dimension_semantics=("arbitrary", "arbitrary")),
    )(x, wrp)


def _ffn_body(te_ref, xs_ref, w1_ref, b1_ref, w2_ref, b2_ref, y_ref):
    h = jnp.dot(xs_ref[...].astype(jnp.bfloat16), w1_ref[0],
                preferred_element_type=jnp.float32) + b1_ref[0]
    h = jax.nn.gelu(h)
    y = jnp.dot(h.astype(jnp.bfloat16), w2_ref[0],
                preferred_element_type=jnp.float32) + b2_ref[0]
    y_ref[...] = y


def _ffn_call(tile_expert, xs, w1b, b1r, w2b, b2r):
    grid_spec = pltpu.PrefetchScalarGridSpec(
        num_scalar_prefetch=1,
        grid=(NTILES,),
        in_specs=[
            pl.BlockSpec((TM, D), lambda i, te: (i, 0)),
            pl.BlockSpec((1, D, F), lambda i, te: (te[i], 0, 0)),
            pl.BlockSpec((1, 1, F), lambda i, te: (te[i], 0, 0)),
            pl.BlockSpec((1, F, D), lambda i, te: (te[i], 0, 0)),
            pl.BlockSpec((1, 1, D), lambda i, te: (te[i], 0, 0)),
        ],
        out_specs=pl.BlockSpec((TM, D), lambda i, te: (i, 0)),
    )
    return pl.pallas_call(
        _ffn_body,
        grid_spec=grid_spec,
        out_shape=jax.ShapeDtypeStruct((NP, D), jnp.float32),
    )(tile_expert, xs, w1b, b1r, w2b, b2r)


def _sc_mesh():
    return plsc.VectorSubcoreMesh(core_axis_name="c", subcore_axis_name="s")


def _scatter_call(x, pos0r, pos1r):
    @functools.partial(
        pl.kernel,
        mesh=_sc_mesh(),
        out_type=jax.ShapeDtypeStruct((NP, D), jnp.float32),
        scratch_types=[
            pltpu.VMEM((CB, D), jnp.float32),
            pltpu.VMEM((CB,), jnp.int32),
            pltpu.VMEM((CB,), jnp.int32),
            pltpu.SemaphoreType.DMA,
            pltpu.SemaphoreType.DMA,
        ],
    )
    def scatter_kernel(x_hbm, pos0_hbm, pos1_hbm, xs_hbm,
                       rows_v, idx0_v, idx1_v, sem0, sem1):
        wid = lax.axis_index("s") * 2 + lax.axis_index("c")
        for c in range(TPW // CB):
            base = wid * TPW + c * CB
            pltpu.sync_copy(x_hbm.at[pl.ds(base, CB)], rows_v)
            pltpu.sync_copy(pos0_hbm.at[wid, c], idx0_v)
            pltpu.sync_copy(pos1_hbm.at[wid, c], idx1_v)
            cp0 = pltpu.async_copy(rows_v, xs_hbm.at[idx0_v], sem0)
            cp1 = pltpu.async_copy(rows_v, xs_hbm.at[idx1_v], sem1)
            cp0.wait()
            cp1.wait()

    return scatter_kernel(x, pos0r, pos1r)


def _combine_call(ys, pos0r, pos1r, g0r, g1r):
    ncd = TPW // CD

    @functools.partial(
        pl.kernel,
        mesh=_sc_mesh(),
        out_type=jax.ShapeDtypeStruct((T, D), jnp.float32),
        scratch_types=[
            pltpu.VMEM((2, CD, D), jnp.float32),
            pltpu.VMEM((2, CD, D), jnp.float32),
            pltpu.VMEM((1, CD, D), jnp.float32),
            pltpu.VMEM((TPW,), jnp.int32),
            pltpu.VMEM((TPW,), jnp.int32),
            pltpu.VMEM((TPW * 16,), jnp.float32),
            pltpu.VMEM((TPW * 16,), jnp.float32),
            pltpu.SemaphoreType.DMA,
            pltpu.SemaphoreType.DMA,
            pltpu.SemaphoreType.DMA,
            pltpu.SemaphoreType.DMA,
        ],
    )
    def combine_kernel(y_hbm, pos0_hbm, pos1_hbm, g0_hbm, g1_hbm, out_hbm,
                       y0_v, y1_v, o_v, idx0_v, idx1_v, g0_v, g1_v,
                       sg0, sg1, so0, so1):
        wid = lax.axis_index("s") * 2 + lax.axis_index("c")
        sg = [sg0, sg1]
        so = [so0, so1]
        handles = {}
        pltpu.sync_copy(pos0_hbm.at[wid], idx0_v)
        pltpu.sync_copy(pos1_hbm.at[wid], idx1_v)
        pltpu.sync_copy(g0_hbm.at[wid], g0_v)
        pltpu.sync_copy(g1_hbm.at[wid], g1_v)

        def issue(c):
            b = c % 2
            handles[(0, c)] = pltpu.async_copy(
                y_hbm.at[idx0_v.at[pl.ds(c * CD, CD)]], y0_v.at[b], sg[b])
            handles[(1, c)] = pltpu.async_copy(
                y_hbm.at[idx1_v.at[pl.ds(c * CD, CD)]], y1_v.at[b], sg[b])

        issue(0)
        for c in range(ncd):
            b = c % 2
            if c + 1 < ncd:
                issue(c + 1)
            handles[(0, c)].wait()
            handles[(1, c)].wait()
            if c >= 1:
                handles[("o", c - 1)].wait()

            def row_body(r, carry):
                g0s = g0_v[pl.ds((c * CD + r) * 16, 16)]
                g1s = g1_v[pl.ds((c * CD + r) * 16, 16)]

                def sl_body(sl, carry2):
                    off = sl * 16
                    o_v[0, r, pl.ds(off, 16)] = (
                        y0_v[b, r, pl.ds(off, 16)] * g0s
                        + y1_v[b, r, pl.ds(off, 16)] * g1s)
                    return carry2

                return lax.fori_loop(0, D // 16, sl_body, carry, unroll=16)

            lax.fori_loop(0, CD, row_body, 0)
            handles[("o", c)] = pltpu.async_copy(
                o_v.at[0], out_hbm.at[pl.ds(wid * TPW + c * CD, CD)], so[b])
        handles[("o", ncd - 1)].wait()

    return combine_kernel(ys, pos0r, pos1r, g0r, g1r)


def kernel(x, w_router, w1, b1, w2, b2):
    wrp = jnp.pad(w_router, ((0, 0), (0, LANES - E)))
    pos, gb, te = _router_call(x, wrp)
    pos0 = pos[:, 0]
    pos1 = pos[:, 64]
    tile_expert = te[:NTILES, 0]
    xs = _scatter_call(
        x,
        pos0.reshape(NW, TPW // CB, CB),
        pos1.reshape(NW, TPW // CB, CB),
    )
    ys = _ffn_call(tile_expert, xs,
                   w1.astype(jnp.bfloat16), b1.reshape(E, 1, F),
                   w2.astype(jnp.bfloat16), b2.reshape(E, 1, D))
    out = _combine_call(
        ys,
        pos0.reshape(NW, TPW),
        pos1.reshape(NW, TPW),
        gb[:, 0:16].reshape(NW, TPW * 16),
        gb[:, 64:80].reshape(NW, TPW * 16),
    )
    return out


# speedup vs baseline: 1.0523x; 1.0185x over previous
import functools

import jax
import jax.numpy as jnp
from jax import lax
from jax.experimental import pallas as pl
from jax.experimental.pallas import tpu as pltpu
from jax.experimental.pallas import tpu_sc as plsc

E = 8
K = 2
D = 1024
F = 4096
T = 8192
TM = 256
NP = T * K + E * TM
NTILES = NP // TM
TILE_T = 512
NT = T // TILE_T
LANES = 128
NW = 32
CB = 64
CD = 16
TPW = T // NW


def _router_body(x_ref, wr_ref, pos_ref, gb_ref, te_ref, cnt_ref, off_ref):
    p = pl.program_id(0)
    i = pl.program_id(1)

    @pl.when(jnp.logical_and(p == 0, i == 0))
    def _init():
        cnt_ref[...] = jnp.zeros_like(cnt_ref)

    @pl.when(jnp.logical_and(p == 1, i == 0))
    def _offsets():
        counts = cnt_ref[0:1, :]
        padded = jnp.floor((counts + (TM - 1)) * (1.0 / TM)) * TM
        ci = lax.broadcasted_iota(jnp.int32, (LANES, LANES), 0)
        cj = lax.broadcasted_iota(jnp.int32, (LANES, LANES), 1)
        upper = (ci < cj).astype(jnp.float32)
        off = jax.lax.dot_general(
            padded, upper, (((1,), (0,)), ((), ())),
            precision=jax.lax.Precision.HIGHEST,
            preferred_element_type=jnp.float32)
        ends = off + padded
        ones_col = jnp.ones((LANES, 1), jnp.float32)
        ends_bc = jax.lax.dot_general(
            ones_col, ends, (((1,), (0,)), ((), ())),
            precision=jax.lax.Precision.HIGHEST,
            preferred_element_type=jnp.float32)
        jrow = (lax.broadcasted_iota(jnp.int32, (LANES, LANES), 0) * TM
                ).astype(jnp.float32)
        ecol = lax.broadcasted_iota(jnp.int32, (LANES, LANES), 1)
        ind = jnp.where((ends_bc <= jrow) & (ecol < E), 1.0, 0.0)
        texp = jnp.minimum(jnp.sum(ind, axis=1, keepdims=True), E - 1.0)
        unused = jnp.where(jrow[:, E - 1:E] < ends_bc[:, E - 1:E], 0.0, 8.0)
        te_ref[...] = jnp.broadcast_to(texp + unused + 0.5,
                                       (LANES, LANES)).astype(jnp.int32)
        off_ref[...] = jnp.broadcast_to(off, (8, LANES))
        cnt_ref[...] = jnp.zeros_like(cnt_ref)

    x = x_ref[...]
    wr = wr_ref[...]
    logits = jnp.dot(x, wr, preferred_element_type=jnp.float32)
    lane = lax.broadcasted_iota(jnp.int32, (TILE_T, LANES), 1)
    valid = lane < E
    lm = jnp.where(valid, logits, -1e30)
    m = jnp.max(lm, axis=1, keepdims=True)
    ez = jnp.where(valid, jnp.exp(lm - m), 0.0)
    probs = ez / jnp.sum(ez, axis=1, keepdims=True)
    p1 = jnp.max(probs, axis=1, keepdims=True)
    i1 = jnp.min(jnp.where((probs == p1) & valid, lane, LANES), axis=1, keepdims=True)
    oh0 = (lane == i1)
    pr2 = jnp.where(oh0, -1.0, probs)
    p2 = jnp.max(pr2, axis=1, keepdims=True)
    i2 = jnp.min(jnp.where((pr2 == p2) & valid, lane, LANES), axis=1, keepdims=True)
    oh1 = (lane == i2)
    s = p1 + p2 + 1e-9
    g0 = p1 / s
    g1 = p2 / s

    oh0f = oh0.astype(jnp.float32)
    oh1f = oh1.astype(jnp.float32)
    ri = lax.broadcasted_iota(jnp.int32, (TILE_T, TILE_T), 0)
    rj = lax.broadcasted_iota(jnp.int32, (TILE_T, TILE_T), 1)
    ltri = (ri >= rj).astype(jnp.float32)
    c0 = jnp.dot(ltri, oh0f, preferred_element_type=jnp.float32)
    c1 = jnp.dot(ltri, oh1f, preferred_element_type=jnp.float32)
    t0 = c0[TILE_T - 1:TILE_T, :]
    t1 = c1[TILE_T - 1:TILE_T, :]
    cnt_row = cnt_ref[0:1, :]

    @pl.when(p == 1)
    def _emit():
        off_row = off_ref[0:1, :]
        c0_at = jnp.sum(c0 * oh0f, axis=1, keepdims=True)
        c1_at = jnp.sum(c1 * oh1f, axis=1, keepdims=True)
        cnt0 = jnp.sum(cnt_row * oh0f, axis=1, keepdims=True)
        cnt1 = jnp.sum(cnt_row * oh1f, axis=1, keepdims=True)
        t0_at1 = jnp.sum(t0 * oh1f, axis=1, keepdims=True)
        off0 = jnp.sum(off_row * oh0f, axis=1, keepdims=True)
        off1 = jnp.sum(off_row * oh1f, axis=1, keepdims=True)
        pos0 = off0 + cnt0 + c0_at - 1.0
        pos1 = off1 + cnt1 + t0_at1 + c1_at - 1.0
        posv = jnp.where(lane < 64, pos0, pos1)
        pos_ref[...] = (posv + 0.5).astype(jnp.int32)
        gb_ref[...] = jnp.where(lane < 64, g0, g1)

    cnt_ref[...] = jnp.broadcast_to(cnt_row + t0 + t1, (8, LANES))


def _router_call(x, wrp):
    return pl.pallas_call(
        _router_body,
        grid=(2, NT),
        in_specs=[
            pl.BlockSpec((TILE_T, D), lambda p, i: (i, 0)),
            pl.BlockSpec((D, LANES), lambda p, i: (0, 0)),
        ],
        out_specs=[
            pl.BlockSpec((TILE_T, LANES), lambda p, i: (i, 0)),
            pl.BlockSpec((TILE_T, LANES), lambda p, i: (i, 0)),
            pl.BlockSpec((LANES, LANES), lambda p, i: (0, 0)),
        ],
        out_shape=[
            jax.ShapeDtypeStruct((T, LANES), jnp.int32),
            jax.ShapeDtypeStruct((T, LANES), jnp.float32),
            jax.ShapeDtypeStruct((LANES, LANES), jnp.int32),
        ],
        scratch_shapes=[
            pltpu.VMEM((8, LANES), jnp.float32),
            pltpu.VMEM((8, LANES), jnp.float32),
        ],
        compiler_params=pltpu.CompilerParams(
            dimension_semantics=("arbitrary", "arbitrary")),
    )(x, wrp)


def _ffn_body(te_ref, xs_ref, w1_ref, b1_ref, w2_ref, b2_ref, y_ref):
    @pl.when(te_ref[pl.program_id(0)] < E)
    def _compute():
        h = jnp.dot(xs_ref[...].astype(jnp.bfloat16), w1_ref[0],
                    preferred_element_type=jnp.float32) + b1_ref[0]
        h = jax.nn.gelu(h)
        y = jnp.dot(h.astype(jnp.bfloat16), w2_ref[0],
                    preferred_element_type=jnp.float32) + b2_ref[0]
        y_ref[...] = y


def _ffn_call(tile_expert, xs, w1b, b1r, w2b, b2r):
    grid_spec = pltpu.PrefetchScalarGridSpec(
        num_scalar_prefetch=1,
        grid=(NTILES,),
        in_specs=[
            pl.BlockSpec((TM, D), lambda i, te: (i, 0)),
            pl.BlockSpec((1, D, F), lambda i, te: (te[i] % E, 0, 0)),
            pl.BlockSpec((1, 1, F), lambda i, te: (te[i] % E, 0, 0)),
            pl.BlockSpec((1, F, D), lambda i, te: (te[i] % E, 0, 0)),
            pl.BlockSpec((1, 1, D), lambda i, te: (te[i] % E, 0, 0)),
        ],
        out_specs=pl.BlockSpec((TM, D), lambda i, te: (i, 0)),
    )
    return pl.pallas_call(
        _ffn_body,
        grid_spec=grid_spec,
        out_shape=jax.ShapeDtypeStruct((NP, D), jnp.float32),
    )(tile_expert, xs, w1b, b1r, w2b, b2r)


def _sc_mesh():
    return plsc.VectorSubcoreMesh(core_axis_name="c", subcore_axis_name="s")


def _scatter_call(x, pos0r, pos1r):
    @functools.partial(
        pl.kernel,
        mesh=_sc_mesh(),
        out_type=jax.ShapeDtypeStruct((NP, D), jnp.float32),
        scratch_types=[
            pltpu.VMEM((CB, D), jnp.float32),
            pltpu.VMEM((CB,), jnp.int32),
            pltpu.VMEM((CB,), jnp.int32),
            pltpu.SemaphoreType.DMA,
            pltpu.SemaphoreType.DMA,
        ],
    )
    def scatter_kernel(x_hbm, pos0_hbm, pos1_hbm, xs_hbm,
                       rows_v, idx0_v, idx1_v, sem0, sem1):
        wid = lax.axis_index("s") * 2 + lax.axis_index("c")
        for c in range(TPW // CB):
            base = wid * TPW + c * CB
            pltpu.sync_copy(x_hbm.at[pl.ds(base, CB)], rows_v)
            pltpu.sync_copy(pos0_hbm.at[wid, c], idx0_v)
            pltpu.sync_copy(pos1_hbm.at[wid, c], idx1_v)
            cp0 = pltpu.async_copy(rows_v, xs_hbm.at[idx0_v], sem0)
            cp1 = pltpu.async_copy(rows_v, xs_hbm.at[idx1_v], sem1)
            cp0.wait()
            cp1.wait()

    return scatter_kernel(x, pos0r, pos1r)


def _combine_call(ys, pos0r, pos1r, g0r, g1r):
    ncd = TPW // CD

    @functools.partial(
        pl.kernel,
        mesh=_sc_mesh(),
        out_type=jax.ShapeDtypeStruct((T, D), jnp.float32),
        scratch_types=[
            pltpu.VMEM((2, CD, D), jnp.float32),
            pltpu.VMEM((2, CD, D), jnp.float32),
            pltpu.VMEM((1, CD, D), jnp.float32),
            pltpu.VMEM((TPW,), jnp.int32),
            pltpu.VMEM((TPW,), jnp.int32),
            pltpu.VMEM((TPW * 16,), jnp.float32),
            pltpu.VMEM((TPW * 16,), jnp.float32),
            pltpu.SemaphoreType.DMA,
            pltpu.SemaphoreType.DMA,
            pltpu.SemaphoreType.DMA,
            pltpu.SemaphoreType.DMA,
        ],
    )
    def combine_kernel(y_hbm, pos0_hbm, pos1_hbm, g0_hbm, g1_hbm, out_hbm,
                       y0_v, y1_v, o_v, idx0_v, idx1_v, g0_v, g1_v,
                       sg0, sg1, so0, so1):
        wid = lax.axis_index("s") * 2 + lax.axis_index("c")
        sg = [sg0, sg1]
        so = [so0, so1]
        handles = {}
        pltpu.sync_copy(pos0_hbm.at[wid], idx0_v)
        pltpu.sync_copy(pos1_hbm.at[wid], idx1_v)
        pltpu.sync_copy(g0_hbm.at[wid], g0_v)
        pltpu.sync_copy(g1_hbm.at[wid], g1_v)

        def issue(c):
            b = c % 2
            handles[(0, c)] = pltpu.async_copy(
                y_hbm.at[idx0_v.at[pl.ds(c * CD, CD)]], y0_v.at[b], sg[b])
            handles[(1, c)] = pltpu.async_copy(
                y_hbm.at[idx1_v.at[pl.ds(c * CD, CD)]], y1_v.at[b], sg[b])

        issue(0)
        for c in range(ncd):
            b = c % 2
            if c + 1 < ncd:
                issue(c + 1)
            handles[(0, c)].wait()
            handles[(1, c)].wait()
            if c >= 1:
                handles[("o", c - 1)].wait()

            def row_body(r, carry):
                g0s = g0_v[pl.ds((c * CD + r) * 16, 16)]
                g1s = g1_v[pl.ds((c * CD + r) * 16, 16)]

                def sl_body(sl, carry2):
                    off = sl * 16
                    o_v[0, r, pl.ds(off, 16)] = (
                        y0_v[b, r, pl.ds(off, 16)] * g0s
                        + y1_v[b, r, pl.ds(off, 16)] * g1s)
                    return carry2

                return lax.fori_loop(0, D // 16, sl_body, carry, unroll=16)

            lax.fori_loop(0, CD, row_body, 0)
            handles[("o", c)] = pltpu.async_copy(
                o_v.at[0], out_hbm.at[pl.ds(wid * TPW + c * CD, CD)], so[b])
        handles[("o", ncd - 1)].wait()

    return combine_kernel(ys, pos0r, pos1r, g0r, g1r)


def kernel(x, w_router, w1, b1, w2, b2):
    wrp = jnp.pad(w_router, ((0, 0), (0, LANES - E)))
    pos, gb, te = _router_call(x, wrp)
    pos0 = pos[:, 0]
    pos1 = pos[:, 64]
    tile_expert = te[:NTILES, 0]
    xs = _scatter_call(
        x,
        pos0.reshape(NW, TPW // CB, CB),
        pos1.reshape(NW, TPW // CB, CB),
    )
    ys = _ffn_call(tile_expert, xs,
                   w1.astype(jnp.bfloat16), b1.reshape(E, 1, F),
                   w2.astype(jnp.bfloat16), b2.reshape(E, 1, D))
    out = _combine_call(
        ys,
        pos0.reshape(NW, TPW),
        pos1.reshape(NW, TPW),
        gb[:, 0:16].reshape(NW, TPW * 16),
        gb[:, 64:80].reshape(NW, TPW * 16),
    )
    return out


# speedup vs baseline: 1.0781x; 1.0245x over previous
import functools

import jax
import jax.numpy as jnp
from jax import lax
from jax.experimental import pallas as pl
from jax.experimental.pallas import tpu as pltpu
from jax.experimental.pallas import tpu_sc as plsc

E = 8
K = 2
D = 1024
F = 4096
T = 8192
TM = 256
NP = T * K + E * TM
NTILES = NP // TM
TILE_T = 512
NT = T // TILE_T
LANES = 128
NW = 32
CB = 64
CD = 16
TPW = T // NW


def _router_body(x_ref, wr_ref, pos_ref, gb_ref, te_ref, cnt_ref, off_ref,
                 cache_ref):
    p = pl.program_id(0)
    i = pl.program_id(1)

    @pl.when(jnp.logical_and(p == 0, i == 0))
    def _init():
        cnt_ref[...] = jnp.zeros_like(cnt_ref)

    @pl.when(jnp.logical_and(p == 1, i == 0))
    def _offsets():
        counts = cnt_ref[0:1, :]
        padded = jnp.floor((counts + (TM - 1)) * (1.0 / TM)) * TM
        ci = lax.broadcasted_iota(jnp.int32, (LANES, LANES), 0)
        cj = lax.broadcasted_iota(jnp.int32, (LANES, LANES), 1)
        upper = (ci < cj).astype(jnp.float32)
        off = jax.lax.dot_general(
            padded, upper, (((1,), (0,)), ((), ())),
            precision=jax.lax.Precision.HIGHEST,
            preferred_element_type=jnp.float32)
        ends = off + padded
        ones_col = jnp.ones((LANES, 1), jnp.float32)
        ends_bc = jax.lax.dot_general(
            ones_col, ends, (((1,), (0,)), ((), ())),
            precision=jax.lax.Precision.HIGHEST,
            preferred_element_type=jnp.float32)
        jrow = (lax.broadcasted_iota(jnp.int32, (LANES, LANES), 0) * TM
                ).astype(jnp.float32)
        ecol = lax.broadcasted_iota(jnp.int32, (LANES, LANES), 1)
        ind = jnp.where((ends_bc <= jrow) & (ecol < E), 1.0, 0.0)
        texp = jnp.minimum(jnp.sum(ind, axis=1, keepdims=True), E - 1.0)
        unused = jnp.where(jrow[:, E - 1:E] < ends_bc[:, E - 1:E], 0.0, 8.0)
        te_ref[...] = jnp.broadcast_to(texp + unused + 0.5,
                                       (LANES, LANES)).astype(jnp.int32)
        off_ref[...] = jnp.broadcast_to(off, (8, LANES))
        cnt_ref[...] = jnp.zeros_like(cnt_ref)

    lane = lax.broadcasted_iota(jnp.int32, (TILE_T, LANES), 1)

    @pl.when(p == 0)
    def _pass0():
        x = x_ref[...]
        wr = wr_ref[...]
        logits = jnp.dot(x, wr, preferred_element_type=jnp.float32)
        valid = lane < E
        lm = jnp.where(valid, logits, -1e30)
        m = jnp.max(lm, axis=1, keepdims=True)
        ez = jnp.where(valid, jnp.exp(lm - m), 0.0)
        probs = ez / jnp.sum(ez, axis=1, keepdims=True)
        p1 = jnp.max(probs, axis=1, keepdims=True)
        i1 = jnp.min(jnp.where((probs == p1) & valid, lane, LANES),
                     axis=1, keepdims=True)
        oh0 = (lane == i1)
        pr2 = jnp.where(oh0, -1.0, probs)
        p2 = jnp.max(pr2, axis=1, keepdims=True)
        i2 = jnp.min(jnp.where((pr2 == p2) & valid, lane, LANES),
                     axis=1, keepdims=True)
        oh1 = (lane == i2)
        s = p1 + p2 + 1e-9
        g0 = p1 / s
        g1 = p2 / s

        oh0f = oh0.astype(jnp.float32)
        oh1f = oh1.astype(jnp.float32)
        ri = lax.broadcasted_iota(jnp.int32, (TILE_T, TILE_T), 0)
        rj = lax.broadcasted_iota(jnp.int32, (TILE_T, TILE_T), 1)
        ltri = (ri >= rj).astype(jnp.float32)
        c0 = jnp.dot(ltri, oh0f, preferred_element_type=jnp.float32)
        c1 = jnp.dot(ltri, oh1f, preferred_element_type=jnp.float32)
        t0 = c0[TILE_T - 1:TILE_T, :]
        t1 = c1[TILE_T - 1:TILE_T, :]
        cnt_row = cnt_ref[0:1, :]
        c0_at = jnp.sum(c0 * oh0f, axis=1, keepdims=True)
        c1_at = jnp.sum(c1 * oh1f, axis=1, keepdims=True)
        cnt0 = jnp.sum(cnt_row * oh0f, axis=1, keepdims=True)
        cnt1 = jnp.sum(cnt_row * oh1f, axis=1, keepdims=True)
        t0_at1 = jnp.sum(t0 * oh1f, axis=1, keepdims=True)
        prepos0 = cnt0 + c0_at - 1.0
        prepos1 = cnt1 + t0_at1 + c1_at - 1.0
        e0f = i1.astype(jnp.float32)
        e1f = i2.astype(jnp.float32)
        cache = jnp.where(lane == 0, prepos0,
                jnp.where(lane == 1, prepos1,
                jnp.where(lane == 2, e0f,
                jnp.where(lane == 3, e1f,
                jnp.where(lane == 4, g0,
                jnp.where(lane == 5, g1, 0.0))))))
        cache_ref[pl.ds(i * TILE_T, TILE_T), :] = cache
        cnt_ref[...] = jnp.broadcast_to(cnt_row + t0 + t1, (8, LANES))

    @pl.when(p == 1)
    def _pass1():
        cb = cache_ref[pl.ds(i * TILE_T, TILE_T), :]

        def col(k):
            return jnp.sum(cb * (lane == k).astype(jnp.float32),
                           axis=1, keepdims=True)

        prepos0, prepos1, e0f, e1f, g0, g1 = (col(0), col(1), col(2),
                                              col(3), col(4), col(5))
        lanef = lane.astype(jnp.float32)
        oh0f = (lanef == e0f).astype(jnp.float32)
        oh1f = (lanef == e1f).astype(jnp.float32)
        off_row = off_ref[0:1, :]
        off0 = jnp.sum(off_row * oh0f, axis=1, keepdims=True)
        off1 = jnp.sum(off_row * oh1f, axis=1, keepdims=True)
        pos0 = off0 + prepos0
        pos1 = off1 + prepos1
        posv = jnp.where(lane < 64, pos0, pos1)
        pos_ref[...] = (posv + 0.5).astype(jnp.int32)
        gb_ref[...] = jnp.where(lane < 64, g0, g1)


def _router_call(x, wrp):
    return pl.pallas_call(
        _router_body,
        grid=(2, NT),
        in_specs=[
            pl.BlockSpec((TILE_T, D), lambda p, i: (i, 0)),
            pl.BlockSpec((D, LANES), lambda p, i: (0, 0)),
        ],
        out_specs=[
            pl.BlockSpec((TILE_T, LANES), lambda p, i: (i, 0)),
            pl.BlockSpec((TILE_T, LANES), lambda p, i: (i, 0)),
            pl.BlockSpec((LANES, LANES), lambda p, i: (0, 0)),
        ],
        out_shape=[
            jax.ShapeDtypeStruct((T, LANES), jnp.int32),
            jax.ShapeDtypeStruct((T, LANES), jnp.float32),
            jax.ShapeDtypeStruct((LANES, LANES), jnp.int32),
        ],
        scratch_shapes=[
            pltpu.VMEM((8, LANES), jnp.float32),
            pltpu.VMEM((8, LANES), jnp.float32),
            pltpu.VMEM((T, LANES), jnp.float32),
        ],
        compiler_params=pltpu.CompilerParams(
            dimension_semantics=("arbitrary", "arbitrary")),
    )(x, wrp)


def _ffn_body(te_ref, xs_ref, w1_ref, b1_ref, w2_ref, b2_ref, y_ref):
    @pl.when(te_ref[pl.program_id(0)] < E)
    def _compute():
        h = jnp.dot(xs_ref[...].astype(jnp.bfloat16), w1_ref[0],
                    preferred_element_type=jnp.float32) + b1_ref[0]
        h = jax.nn.gelu(h)
        y = jnp.dot(h.astype(jnp.bfloat16), w2_ref[0],
                    preferred_element_type=jnp.float32) + b2_ref[0]
        y_ref[...] = y


def _ffn_call(tile_expert, xs, w1b, b1r, w2b, b2r):
    grid_spec = pltpu.PrefetchScalarGridSpec(
        num_scalar_prefetch=1,
        grid=(NTILES,),
        in_specs=[
            pl.BlockSpec((TM, D), lambda i, te: (i, 0)),
            pl.BlockSpec((1, D, F), lambda i, te: (te[i] % E, 0, 0)),
            pl.BlockSpec((1, 1, F), lambda i, te: (te[i] % E, 0, 0)),
            pl.BlockSpec((1, F, D), lambda i, te: (te[i] % E, 0, 0)),
            pl.BlockSpec((1, 1, D), lambda i, te: (te[i] % E, 0, 0)),
        ],
        out_specs=pl.BlockSpec((TM, D), lambda i, te: (i, 0)),
    )
    return pl.pallas_call(
        _ffn_body,
        grid_spec=grid_spec,
        out_shape=jax.ShapeDtypeStruct((NP, D), jnp.float32),
    )(tile_expert, xs, w1b, b1r, w2b, b2r)


def _sc_mesh():
    return plsc.VectorSubcoreMesh(core_axis_name="c", subcore_axis_name="s")


def _scatter_call(x, pos0r, pos1r):
    @functools.partial(
        pl.kernel,
        mesh=_sc_mesh(),
        out_type=jax.ShapeDtypeStruct((NP, D), jnp.float32),
        scratch_types=[
            pltpu.VMEM((CB, D), jnp.float32),
            pltpu.VMEM((CB,), jnp.int32),
            pltpu.VMEM((CB,), jnp.int32),
            pltpu.SemaphoreType.DMA,
            pltpu.SemaphoreType.DMA,
        ],
    )
    def scatter_kernel(x_hbm, pos0_hbm, pos1_hbm, xs_hbm,
                       rows_v, idx0_v, idx1_v, sem0, sem1):
        wid = lax.axis_index("s") * 2 + lax.axis_index("c")
        for c in range(TPW // CB):
            base = wid * TPW + c * CB
            pltpu.sync_copy(x_hbm.at[pl.ds(base, CB)], rows_v)
            pltpu.sync_copy(pos0_hbm.at[wid, c], idx0_v)
            pltpu.sync_copy(pos1_hbm.at[wid, c], idx1_v)
            cp0 = pltpu.async_copy(rows_v, xs_hbm.at[idx0_v], sem0)
            cp1 = pltpu.async_copy(rows_v, xs_hbm.at[idx1_v], sem1)
            cp0.wait()
            cp1.wait()

    return scatter_kernel(x, pos0r, pos1r)


def _combine_call(ys, pos0r, pos1r, g0r, g1r):
    ncd = TPW // CD

    @functools.partial(
        pl.kernel,
        mesh=_sc_mesh(),
        out_type=jax.ShapeDtypeStruct((T, D), jnp.float32),
        scratch_types=[
            pltpu.VMEM((2, CD, D), jnp.float32),
            pltpu.VMEM((2, CD, D), jnp.float32),
            pltpu.VMEM((1, CD, D), jnp.float32),
            pltpu.VMEM((TPW,), jnp.int32),
            pltpu.VMEM((TPW,), jnp.int32),
            pltpu.VMEM((TPW * 16,), jnp.float32),
            pltpu.VMEM((TPW * 16,), jnp.float32),
            pltpu.SemaphoreType.DMA,
            pltpu.SemaphoreType.DMA,
            pltpu.SemaphoreType.DMA,
            pltpu.SemaphoreType.DMA,
        ],
    )
    def combine_kernel(y_hbm, pos0_hbm, pos1_hbm, g0_hbm, g1_hbm, out_hbm,
                       y0_v, y1_v, o_v, idx0_v, idx1_v, g0_v, g1_v,
                       sg0, sg1, so0, so1):
        wid = lax.axis_index("s") * 2 + lax.axis_index("c")
        sg = [sg0, sg1]
        so = [so0, so1]
        handles = {}
        pltpu.sync_copy(pos0_hbm.at[wid], idx0_v)
        pltpu.sync_copy(pos1_hbm.at[wid], idx1_v)
        pltpu.sync_copy(g0_hbm.at[wid], g0_v)
        pltpu.sync_copy(g1_hbm.at[wid], g1_v)

        def issue(c):
            b = c % 2
            handles[(0, c)] = pltpu.async_copy(
                y_hbm.at[idx0_v.at[pl.ds(c * CD, CD)]], y0_v.at[b], sg[b])
            handles[(1, c)] = pltpu.async_copy(
                y_hbm.at[idx1_v.at[pl.ds(c * CD, CD)]], y1_v.at[b], sg[b])

        issue(0)
        for c in range(ncd):
            b = c % 2
            if c + 1 < ncd:
                issue(c + 1)
            handles[(0, c)].wait()
            handles[(1, c)].wait()
            if c >= 1:
                handles[("o", c - 1)].wait()

            def row_body(r, carry):
                g0s = g0_v[pl.ds((c * CD + r) * 16, 16)]
                g1s = g1_v[pl.ds((c * CD + r) * 16, 16)]

                def sl_body(sl, carry2):
                    off = sl * 16
                    o_v[0, r, pl.ds(off, 16)] = (
                        y0_v[b, r, pl.ds(off, 16)] * g0s
                        + y1_v[b, r, pl.ds(off, 16)] * g1s)
                    return carry2

                return lax.fori_loop(0, D // 16, sl_body, carry, unroll=16)

            lax.fori_loop(0, CD, row_body, 0)
            handles[("o", c)] = pltpu.async_copy(
                o_v.at[0], out_hbm.at[pl.ds(wid * TPW + c * CD, CD)], so[b])
        handles[("o", ncd - 1)].wait()

    return combine_kernel(ys, pos0r, pos1r, g0r, g1r)


def kernel(x, w_router, w1, b1, w2, b2):
    wrp = jnp.pad(w_router, ((0, 0), (0, LANES - E)))
    pos, gb, te = _router_call(x, wrp)
    pos0 = pos[:, 0]
    pos1 = pos[:, 64]
    tile_expert = te[:NTILES, 0]
    xs = _scatter_call(
        x,
        pos0.reshape(NW, TPW // CB, CB),
        pos1.reshape(NW, TPW // CB, CB),
    )
    ys = _ffn_call(tile_expert, xs,
                   w1.astype(jnp.bfloat16), b1.reshape(E, 1, F),
                   w2.astype(jnp.bfloat16), b2.reshape(E, 1, D))
    out = _combine_call(
        ys,
        pos0.reshape(NW, TPW),
        pos1.reshape(NW, TPW),
        gb[:, 0:16].reshape(NW, TPW * 16),
        gb[:, 64:80].reshape(NW, TPW * 16),
    )
    return out
